# baseline (speedup 1.0000x reference)
import functools

import jax
import jax.numpy as jnp
import numpy as np
from jax.experimental import pallas as pl
from jax.experimental.pallas import tpu as pltpu

N = 50000
E = 800000
B = 2500
H = 60
NB = 50
RMAX = 10.0
NLAYERS = 3
NUM_NEIGHBORS = 20.0
NUM_NODES = 20.0

BE = 4000


def _edge_layer_body(geom_ref, xs_ref, fc1_ref, fc1b_ref, fc2_ref, fc2b_ref,
                     fc3_ref, fc3b_ref, wsh_ref, msg_ref):
    geom = geom_ref[...]
    x = geom[:, 0:1]
    y = geom[:, 1:2]
    z = geom[:, 2:3]
    length = geom[:, 3:4]

    s3 = 3.0 ** 0.5
    s5 = 5.0 ** 0.5
    s15 = 15.0 ** 0.5
    sh = jnp.concatenate([
        jnp.ones_like(x),
        s3 * x, s3 * y, s3 * z,
        s15 * x * y, s15 * y * z, (s5 / 2.0) * (3.0 * z * z - 1.0),
        s15 * x * z, (s15 / 2.0) * (x * x - y * y)
    ], axis=1)

    step = RMAX / (NB + 1)
    cols = jax.lax.broadcasted_iota(jnp.int32, (length.shape[0], NB), 1)
    u = length / step - (cols.astype(jnp.float32) + 1.0)
    inside = jnp.abs(u) < 1.0
    denom = jnp.where(inside, u * u - 1.0, -1.0)
    emb = jnp.where(inside, 1.14136 * np.exp(2.0) * jnp.exp(1.0 / denom), 0.0)
    emb = emb * (NB ** 0.5)

    h = emb @ fc1_ref[...] + fc1b_ref[...]
    h = h * jax.nn.sigmoid(h)
    h = h @ fc2_ref[...] + fc2b_ref[...]
    h = h * jax.nn.sigmoid(h)
    we = h @ fc3_ref[...] + fc3b_ref[...]
    shp = sh @ wsh_ref[...]
    msg_ref[...] = xs_ref[...] * shp * we


@functools.partial(jax.jit, static_argnames=())
def _edge_layer(geom, xs, fc1_w, fc1_b, fc2_w, fc2_b, fc3_w, fc3_b, wsh):
    grid = (E // BE,)
    full = lambda shape: pl.BlockSpec(shape, lambda i: (0,) * len(shape))
    return pl.pallas_call(
        _edge_layer_body,
        grid=grid,
        in_specs=[
            pl.BlockSpec((BE, 4), lambda i: (i, 0)),
            pl.BlockSpec((BE, H), lambda i: (i, 0)),
            full((NB, 128)),
            full((1, 128)),
            full((128, 128)),
            full((1, 128)),
            full((128, H)),
            full((1, H)),
            full((9, H)),
        ],
        out_specs=pl.BlockSpec((BE, H), lambda i: (i, 0)),
        out_shape=jax.ShapeDtypeStruct((E, H), jnp.float32),
    )(geom, xs, fc1_w, fc1_b.reshape(1, 128), fc2_w, fc2_b.reshape(1, 128),
      fc3_w, fc3_b.reshape(1, H), wsh)


def kernel(z, pos, batch, edge_src, edge_dst, W_embed, W_attr,
           fc1_w, fc1_b, fc2_w, fc2_b, fc3_w, fc3_b, Wsh, Wself, Wout):
    edge_vec = pos[edge_src] - pos[edge_dst]
    length = jnp.sqrt(jnp.sum(edge_vec * edge_vec, axis=1) + 1e-12)
    unit = edge_vec / length[:, None]
    geom = jnp.concatenate([unit, length[:, None]], axis=1)

    table = jnp.array([-1, 0, -1, -1, -1, -1, 1, 2, 3, 4])
    node_attr = jax.nn.one_hot(table[z], 5, dtype=jnp.float32) * (5.0 ** 0.5)
    ones = jnp.ones((pos.shape[0], 1), jnp.float32)
    x = ones @ W_embed + node_attr @ W_attr

    for k in range(NLAYERS):
        xs = x[edge_src]
        msg = _edge_layer(geom, xs, fc1_w[k], fc1_b[k], fc2_w[k], fc2_b[k],
                          fc3_w[k], fc3_b[k], Wsh[k])
        agg = jax.ops.segment_sum(msg, edge_dst, num_segments=N) / (NUM_NEIGHBORS ** 0.5)
        x = jax.nn.silu((x + agg) @ Wself[k])

    node_out = x @ Wout
    s = node_out[:, 0] + 0.5 * node_out[:, 1] ** 2
    s = (s / (NUM_NODES ** 0.5))[:, None]
    out = jax.ops.segment_sum(s, batch, num_segments=B)
    return out


# speedup vs baseline: 1.0073x; 1.0073x over previous
import functools

import jax
import jax.numpy as jnp
import numpy as np
from jax import lax
from jax.experimental import pallas as pl
from jax.experimental.pallas import tpu as pltpu

N = 50000
E = 800000
B = 2500
H = 60
NB = 50
RMAX = 10.0
NLAYERS = 3
NUM_NEIGHBORS = 20.0
NUM_NODES = 20.0

BE = 4000


def _edge_layer_body(geom_ref, xs_ref, fc1_ref, fc1b_ref, fc2_ref, fc2b_ref,
                     fc3_ref, fc3b_ref, wsh_ref, msg_ref):
    geom = geom_ref[...]
    x = geom[:, 0:1]
    y = geom[:, 1:2]
    z = geom[:, 2:3]
    length = geom[:, 3:4]

    s3 = 3.0 ** 0.5
    s5 = 5.0 ** 0.5
    s15 = 15.0 ** 0.5
    sh = jnp.concatenate([
        jnp.ones_like(x),
        s3 * x, s3 * y, s3 * z,
        s15 * x * y, s15 * y * z, (s5 / 2.0) * (3.0 * z * z - 1.0),
        s15 * x * z, (s15 / 2.0) * (x * x - y * y)
    ], axis=1)

    step = RMAX / (NB + 1)
    cols = jax.lax.broadcasted_iota(jnp.int32, (length.shape[0], NB), 1)
    u = length / step - (cols.astype(jnp.float32) + 1.0)
    inside = jnp.abs(u) < 1.0
    denom = jnp.where(inside, u * u - 1.0, -1.0)
    emb = jnp.where(inside, 1.14136 * np.exp(2.0) * jnp.exp(1.0 / denom), 0.0)
    emb = emb * (NB ** 0.5)

    def mm(a, b):
        return jax.lax.dot(a.astype(jnp.bfloat16), b,
                           preferred_element_type=jnp.float32)

    h = mm(emb, fc1_ref[...]) + fc1b_ref[...]
    h = h * jax.nn.sigmoid(h)
    h = mm(h, fc2_ref[...]) + fc2b_ref[...]
    h = h * jax.nn.sigmoid(h)
    we = mm(h, fc3_ref[...]) + fc3b_ref[...]
    shp = mm(sh, wsh_ref[...])
    msg_ref[...] = xs_ref[...] * shp * we


def _edge_layer(geom, xs, fc1_w, fc1_b, fc2_w, fc2_b, fc3_w, fc3_b, wsh):
    grid = (E // BE,)
    full = lambda shape: pl.BlockSpec(shape, lambda i: (0,) * len(shape))
    return pl.pallas_call(
        _edge_layer_body,
        grid=grid,
        in_specs=[
            pl.BlockSpec((BE, 4), lambda i: (i, 0)),
            pl.BlockSpec((BE, H), lambda i: (i, 0)),
            full((NB, 128)),
            full((1, 128)),
            full((128, 128)),
            full((1, 128)),
            full((128, H)),
            full((1, H)),
            full((9, H)),
        ],
        out_specs=pl.BlockSpec((BE, H), lambda i: (i, 0)),
        out_shape=jax.ShapeDtypeStruct((E, H), jnp.float32),
    )(geom, xs, fc1_w, fc1_b.reshape(1, 128), fc2_w, fc2_b.reshape(1, 128),
      fc3_w, fc3_b.reshape(1, H), wsh)


def kernel(z, pos, batch, edge_src, edge_dst, W_embed, W_attr,
           fc1_w, fc1_b, fc2_w, fc2_b, fc3_w, fc3_b, Wsh, Wself, Wout):
    edge_vec = jnp.take(pos, edge_src, axis=0, mode="clip") - \
        jnp.take(pos, edge_dst, axis=0, mode="clip")
    length = jnp.sqrt(jnp.sum(edge_vec * edge_vec, axis=1) + 1e-12)
    unit = edge_vec / length[:, None]
    geom = jnp.concatenate([unit, length[:, None]], axis=1)

    table = jnp.array([-1, 0, -1, -1, -1, -1, 1, 2, 3, 4])
    node_attr = jax.nn.one_hot(table[z], 5, dtype=jnp.float32) * (5.0 ** 0.5)
    ones = jnp.ones((pos.shape[0], 1), jnp.float32)
    x = ones @ W_embed + node_attr @ W_attr

    fc1_wb = fc1_w.astype(jnp.bfloat16)
    fc2_wb = fc2_w.astype(jnp.bfloat16)
    fc3_wb = fc3_w.astype(jnp.bfloat16)
    wsh_b = Wsh.astype(jnp.bfloat16)

    for k in range(NLAYERS):
        xs = jnp.take(x, edge_src, axis=0, mode="clip")
        msg = _edge_layer(geom, xs, fc1_wb[k], fc1_b[k], fc2_wb[k], fc2_b[k],
                          fc3_wb[k], fc3_b[k], wsh_b[k])
        agg = jax.ops.segment_sum(msg, edge_dst, num_segments=N) / (NUM_NEIGHBORS ** 0.5)
        x = jax.nn.silu((x + agg) @ Wself[k])

    node_out = x @ Wout
    s = node_out[:, 0] + 0.5 * node_out[:, 1] ** 2
    s = (s / (NUM_NODES ** 0.5))[:, None]
    out = jax.ops.segment_sum(s, batch, num_segments=B)
    return out


# speedup vs baseline: 1.0206x; 1.0131x over previous
import functools

import jax
import jax.numpy as jnp
import numpy as np
from jax import lax
from jax.experimental import pallas as pl
from jax.experimental.pallas import tpu as pltpu

N = 50000
E = 800000
B = 2500
H = 60
NB = 50
RMAX = 10.0
NLAYERS = 3
NUM_NEIGHBORS = 20.0
NUM_NODES = 20.0

BE = 4000


def _edge_layer_body(geom_ref, xs_ref, fc1_ref, fc1b_ref, fc2_ref, fc2b_ref,
                     fc3_ref, fc3b_ref, wsh_ref, msg_ref):
    geom = geom_ref[...]
    x = geom[:, 0:1]
    y = geom[:, 1:2]
    z = geom[:, 2:3]
    length = geom[:, 3:4]

    s3 = 3.0 ** 0.5
    s5 = 5.0 ** 0.5
    s15 = 15.0 ** 0.5
    sh = jnp.concatenate([
        jnp.ones_like(x),
        s3 * x, s3 * y, s3 * z,
        s15 * x * y, s15 * y * z, (s5 / 2.0) * (3.0 * z * z - 1.0),
        s15 * x * z, (s15 / 2.0) * (x * x - y * y)
    ], axis=1)

    step = RMAX / (NB + 1)
    cols = jax.lax.broadcasted_iota(jnp.int32, (length.shape[0], NB), 1)
    u = length / step - (cols.astype(jnp.float32) + 1.0)
    inside = jnp.abs(u) < 1.0
    denom = jnp.where(inside, u * u - 1.0, -1.0)
    emb = jnp.where(inside, 1.14136 * np.exp(2.0) * jnp.exp(1.0 / denom), 0.0)
    emb = emb * (NB ** 0.5)

    def mm(a, b):
        return jax.lax.dot(a.astype(jnp.bfloat16), b,
                           preferred_element_type=jnp.float32)

    h = mm(emb, fc1_ref[...]) + fc1b_ref[...]
    h = h * jax.nn.sigmoid(h)
    h = mm(h, fc2_ref[...]) + fc2b_ref[...]
    h = h * jax.nn.sigmoid(h)
    we = mm(h, fc3_ref[...]) + fc3b_ref[...]
    shp = mm(sh, wsh_ref[...])
    msg_ref[...] = xs_ref[...] * shp * we


def _edge_layer(geom, xs, fc1_w, fc1_b, fc2_w, fc2_b, fc3_w, fc3_b, wsh):
    grid = (E // BE,)
    full = lambda shape: pl.BlockSpec(shape, lambda i: (0,) * len(shape))
    return pl.pallas_call(
        _edge_layer_body,
        grid=grid,
        in_specs=[
            pl.BlockSpec((BE, 4), lambda i: (i, 0)),
            pl.BlockSpec((BE, H), lambda i: (i, 0)),
            full((NB, 128)),
            full((1, 128)),
            full((128, 128)),
            full((1, 128)),
            full((128, H)),
            full((1, H)),
            full((9, H)),
        ],
        out_specs=pl.BlockSpec((BE, H), lambda i: (i, 0)),
        out_shape=jax.ShapeDtypeStruct((E, H), jnp.float32),
    )(geom, xs, fc1_w, fc1_b.reshape(1, 128), fc2_w, fc2_b.reshape(1, 128),
      fc3_w, fc3_b.reshape(1, H), wsh)


def kernel(z, pos, batch, edge_src, edge_dst, W_embed, W_attr,
           fc1_w, fc1_b, fc2_w, fc2_b, fc3_w, fc3_b, Wsh, Wself, Wout):
    edge_vec = pos[edge_src] - pos[edge_dst]
    length = jnp.sqrt(jnp.sum(edge_vec * edge_vec, axis=1) + 1e-12)
    unit = edge_vec / length[:, None]
    geom = jnp.concatenate([unit, length[:, None]], axis=1)

    table = jnp.array([-1, 0, -1, -1, -1, -1, 1, 2, 3, 4])
    node_attr = jax.nn.one_hot(table[z], 5, dtype=jnp.float32) * (5.0 ** 0.5)
    ones = jnp.ones((pos.shape[0], 1), jnp.float32)
    x = ones @ W_embed + node_attr @ W_attr

    fc1_wb = fc1_w.astype(jnp.bfloat16)
    fc2_wb = fc2_w.astype(jnp.bfloat16)
    fc3_wb = fc3_w.astype(jnp.bfloat16)
    wsh_b = Wsh.astype(jnp.bfloat16)

    for k in range(NLAYERS):
        xs = x[edge_src]
        msg = _edge_layer(geom, xs, fc1_wb[k], fc1_b[k], fc2_wb[k], fc2_b[k],
                          fc3_wb[k], fc3_b[k], wsh_b[k])
        agg = jax.ops.segment_sum(msg, edge_dst, num_segments=N) / (NUM_NEIGHBORS ** 0.5)
        x = jax.nn.silu((x + agg) @ Wself[k])

    node_out = x @ Wout
    s = node_out[:, 0] + 0.5 * node_out[:, 1] ** 2
    s = (s / (NUM_NODES ** 0.5))[:, None]
    out = jax.ops.segment_sum(s, batch, num_segments=B)
    return out
